# baseline (speedup 1.0000x reference)
import functools

import jax
import jax.numpy as jnp
from jax import lax
from jax.experimental import pallas as pl
from jax.experimental.pallas import tpu as pltpu
from jax.experimental.pallas import tpu_sc as plsc

N = 10000
D = 128
F = 4
G = 64
H = 512
C = 10

NC, NS, L = 2, 16, 16
NW = NC * NS
R = 512
RPW = R // NW
S = N - R

BM = 672
M_BLOCKS = (S + BM - 1) // BM
KCH = N // L


def _xw_kernel(x_ref, wg_ref, xw_ref):
    xw_ref[...] = jnp.dot(x_ref[...], wg_ref[...],
                          preferred_element_type=jnp.float32)


def _sc_rows(filt_hbm, xwt_hbm, out_hbm, rowbuf, xw0, xw1, xw2, xw3, outbuf):
    wid = lax.axis_index("s") * NC + lax.axis_index("c")
    base = S + wid * RPW
    pltpu.sync_copy(xwt_hbm.at[0], xw0)
    pltpu.sync_copy(xwt_hbm.at[1], xw1)
    pltpu.sync_copy(xwt_hbm.at[2], xw2)
    pltpu.sync_copy(xwt_hbm.at[3], xw3)

    def row_body(i, carry):
        pltpu.sync_copy(filt_hbm.at[base + i], rowbuf)

        def chunk(c, accs):
            a0, a1, a2, a3 = accs
            rv = rowbuf[pl.ds(c * L, L)]
            a0 = a0 + rv * xw0[pl.ds(c * L, L)]
            a1 = a1 + rv * xw1[pl.ds(c * L, L)]
            a2 = a2 + rv * xw2[pl.ds(c * L, L)]
            a3 = a3 + rv * xw3[pl.ds(c * L, L)]
            return (a0, a1, a2, a3)

        z = jnp.zeros((L,), jnp.float32)
        a0, a1, a2, a3 = lax.fori_loop(0, KCH, chunk, (z, z, z, z))
        outbuf[pl.ds(i * 4 * L, L)] = a0
        outbuf[pl.ds(i * 4 * L + L, L)] = a1
        outbuf[pl.ds(i * 4 * L + 2 * L, L)] = a2
        outbuf[pl.ds(i * 4 * L + 3 * L, L)] = a3
        return carry

    lax.fori_loop(0, RPW, row_body, 0)
    pltpu.sync_copy(outbuf, out_hbm.at[pl.ds(wid * RPW * 4 * L, RPW * 4 * L)])


def _body_kernel(xw_ref, filt_ref, ids_ref, pool_ref, pooled_ref):
    m = pl.program_id(0)

    @pl.when(m == 0)
    def _init():
        pooled_ref[...] = jnp.zeros_like(pooled_ref)

    h_blk = jnp.maximum(
        jnp.dot(filt_ref[...], xw_ref[...],
                preferred_element_type=jnp.float32), 0.0)
    gids = jax.lax.broadcasted_iota(jnp.int32, (BM, G), 1)
    rows = jax.lax.broadcasted_iota(jnp.int32, (BM, G), 0) + m * BM
    oh = (ids_ref[...] == gids) & (rows < N)
    cols = [jnp.max(jnp.where(oh, h_blk[:, f:f + 1], 0.0),
                    axis=0, keepdims=True) for f in range(F)]
    local = jnp.concatenate(cols, axis=0)
    pooled_ref[...] = jnp.maximum(pooled_ref[...], local)

    @pl.when(m == M_BLOCKS - 1)
    def _out():
        pool_ref[...] = pooled_ref[...]


def _merge_kernel(ptc_ref, part_ref, ids_ref, wh_ref, bh_ref,
                  wc_ref, bc_ref, out_ref):
    jrow = jax.lax.broadcasted_iota(jnp.int32, (4 * L, F), 0) // L
    jcol = jax.lax.broadcasted_iota(jnp.int32, (4 * L, F), 1)
    sel = jnp.where(jrow == jcol, 1.0, 0.0)
    h_sc = jnp.maximum(
        jnp.dot(part_ref[...], sel,
                preferred_element_type=jnp.float32), 0.0)
    gids = jax.lax.broadcasted_iota(jnp.int32, (R, G), 1)
    oh = ids_ref[...] == gids
    cols = [jnp.max(jnp.where(oh, h_sc[:, f:f + 1], 0.0),
                    axis=0, keepdims=True) for f in range(F)]
    local = jnp.concatenate(cols, axis=0)
    pooled_t = jnp.maximum(ptc_ref[...], local)
    z = jnp.maximum(
        jax.lax.dot_general(pooled_t, wh_ref[...],
                            (((0,), (0,)), ((), ())),
                            preferred_element_type=jnp.float32)
        + bh_ref[...], 0.0)
    logits = jnp.dot(z, wc_ref[...],
                     preferred_element_type=jnp.float32) + bc_ref[...]
    mx = jnp.max(logits, axis=-1, keepdims=True)
    e = jnp.exp(logits - mx)
    out_ref[...] = e / jnp.sum(e, axis=-1, keepdims=True)


_sc_call = functools.partial(
    pl.kernel,
    mesh=plsc.VectorSubcoreMesh(core_axis_name="c", subcore_axis_name="s"),
    out_type=jax.ShapeDtypeStruct((R * 4 * L,), jnp.float32),
    scratch_types=[
        pltpu.VMEM((N,), jnp.float32),
        pltpu.VMEM((N,), jnp.float32),
        pltpu.VMEM((N,), jnp.float32),
        pltpu.VMEM((N,), jnp.float32),
        pltpu.VMEM((N,), jnp.float32),
        pltpu.VMEM((RPW * 4 * L,), jnp.float32),
    ],
)


@jax.jit
def _run(filtre, X, ids2, ids_sc, W_gcn, W_h, b_h, W_c, b_c):
    xw = pl.pallas_call(
        _xw_kernel,
        out_shape=jax.ShapeDtypeStruct((N, F), jnp.float32),
    )(X, W_gcn)
    xwt = xw.T

    partials = _sc_call(_sc_rows)(filtre, xwt)

    pooled_tc = pl.pallas_call(
        _body_kernel,
        grid=(M_BLOCKS,),
        in_specs=[
            pl.BlockSpec((N, F), lambda m: (0, 0)),
            pl.BlockSpec((BM, N), lambda m: (m, 0)),
            pl.BlockSpec((BM, 1), lambda m: (m, 0)),
        ],
        out_specs=pl.BlockSpec((F, G), lambda m: (0, 0)),
        out_shape=jax.ShapeDtypeStruct((F, G), jnp.float32),
        compiler_params=pltpu.CompilerParams(
            vmem_limit_bytes=64 * 1024 * 1024),
        scratch_shapes=[
            pltpu.VMEM((F, G), jnp.float32),
        ],
    )(xw, filtre, ids2)

    return pl.pallas_call(
        _merge_kernel,
        out_shape=jax.ShapeDtypeStruct((G, C), jnp.float32),
    )(pooled_tc, partials.reshape(R, 4 * L), ids_sc, W_h, b_h, W_c, b_c)


def kernel(filtre, X, node_indicator, W_gcn, W_h, b_h, W_c, b_c):
    ids2 = node_indicator.astype(jnp.int32).reshape(N, 1)
    return _run(filtre, X, ids2, ids2[S:], W_gcn, W_h,
                b_h.reshape(1, H), W_c, b_c.reshape(1, C))


# speedup vs baseline: 1.0239x; 1.0239x over previous
import functools

import jax
import jax.numpy as jnp
from jax import lax
from jax.experimental import pallas as pl
from jax.experimental.pallas import tpu as pltpu
from jax.experimental.pallas import tpu_sc as plsc

N = 10000
D = 128
F = 4
G = 64
H = 512
C = 10

NC, NS, L = 2, 16, 16
NW = NC * NS
R = 640
RPW = R // NW
S = N - R

BM = 672
M_BLOCKS = (S + BM - 1) // BM
UNROLL = 5
KCH5 = N // (L * UNROLL)


def _xw_kernel(x_ref, wg_ref, xw_ref):
    xw_ref[...] = jnp.dot(x_ref[...], wg_ref[...],
                          preferred_element_type=jnp.float32)


def _sc_rows(filt_hbm, xwt_hbm, out_hbm, bufa, bufb, xw0, xw1, xw2, xw3,
             outbuf, sema, semb):
    wid = lax.axis_index("s") * NC + lax.axis_index("c")
    base = S + wid * RPW
    pltpu.sync_copy(xwt_hbm.at[0], xw0)
    pltpu.sync_copy(xwt_hbm.at[1], xw1)
    pltpu.sync_copy(xwt_hbm.at[2], xw2)
    pltpu.sync_copy(xwt_hbm.at[3], xw3)

    def row_dot(buf, i):
        def chunk(ci, accs):
            a0, a1, a2, a3 = accs
            for u in range(UNROLL):
                off = ci * (L * UNROLL) + u * L
                rv = buf[pl.ds(off, L)]
                a0 = a0 + rv * xw0[pl.ds(off, L)]
                a1 = a1 + rv * xw1[pl.ds(off, L)]
                a2 = a2 + rv * xw2[pl.ds(off, L)]
                a3 = a3 + rv * xw3[pl.ds(off, L)]
            return (a0, a1, a2, a3)

        z = jnp.zeros((L,), jnp.float32)
        a0, a1, a2, a3 = lax.fori_loop(0, KCH5, chunk, (z, z, z, z))
        outbuf[pl.ds(i * 4 * L, L)] = a0
        outbuf[pl.ds(i * 4 * L + L, L)] = a1
        outbuf[pl.ds(i * 4 * L + 2 * L, L)] = a2
        outbuf[pl.ds(i * 4 * L + 3 * L, L)] = a3

    pltpu.async_copy(filt_hbm.at[base], bufa, sema)

    def pair(j, carry):
        r0 = base + 2 * j
        pltpu.async_copy(filt_hbm.at[r0 + 1], bufb, semb)
        pltpu.make_async_copy(filt_hbm.at[r0], bufa, sema).wait()
        row_dot(bufa, 2 * j)
        pltpu.async_copy(
            filt_hbm.at[jnp.minimum(r0 + 2, N - 1)], bufa, sema)
        pltpu.make_async_copy(filt_hbm.at[r0 + 1], bufb, semb).wait()
        row_dot(bufb, 2 * j + 1)
        return carry

    lax.fori_loop(0, RPW // 2, pair, 0)
    pltpu.make_async_copy(filt_hbm.at[0], bufa, sema).wait()
    pltpu.sync_copy(outbuf, out_hbm.at[pl.ds(wid * RPW * 4 * L, RPW * 4 * L)])


def _body_kernel(xw_ref, filt_ref, ids_ref, pool_ref, pooled_ref):
    m = pl.program_id(0)

    @pl.when(m == 0)
    def _init():
        pooled_ref[...] = jnp.zeros_like(pooled_ref)

    h_blk = jnp.maximum(
        jnp.dot(filt_ref[...], xw_ref[...],
                preferred_element_type=jnp.float32), 0.0)
    gids = jax.lax.broadcasted_iota(jnp.int32, (BM, G), 1)
    rows = jax.lax.broadcasted_iota(jnp.int32, (BM, G), 0) + m * BM
    oh = (ids_ref[...] == gids) & (rows < N)
    cols = [jnp.max(jnp.where(oh, h_blk[:, f:f + 1], 0.0),
                    axis=0, keepdims=True) for f in range(F)]
    local = jnp.concatenate(cols, axis=0)
    pooled_ref[...] = jnp.maximum(pooled_ref[...], local)

    @pl.when(m == M_BLOCKS - 1)
    def _out():
        pool_ref[...] = pooled_ref[...]


def _merge_kernel(ptc_ref, part_ref, ids_ref, wh_ref, bh_ref,
                  wc_ref, bc_ref, out_ref):
    jrow = jax.lax.broadcasted_iota(jnp.int32, (4 * L, F), 0) // L
    jcol = jax.lax.broadcasted_iota(jnp.int32, (4 * L, F), 1)
    sel = jnp.where(jrow == jcol, 1.0, 0.0)
    h_sc = jnp.maximum(
        jnp.dot(part_ref[...], sel,
                preferred_element_type=jnp.float32), 0.0)
    gids = jax.lax.broadcasted_iota(jnp.int32, (R, G), 1)
    oh = ids_ref[...] == gids
    cols = [jnp.max(jnp.where(oh, h_sc[:, f:f + 1], 0.0),
                    axis=0, keepdims=True) for f in range(F)]
    local = jnp.concatenate(cols, axis=0)
    pooled_t = jnp.maximum(ptc_ref[...], local)
    z = jnp.maximum(
        jax.lax.dot_general(pooled_t, wh_ref[...],
                            (((0,), (0,)), ((), ())),
                            preferred_element_type=jnp.float32)
        + bh_ref[...], 0.0)
    logits = jnp.dot(z, wc_ref[...],
                     preferred_element_type=jnp.float32) + bc_ref[...]
    mx = jnp.max(logits, axis=-1, keepdims=True)
    e = jnp.exp(logits - mx)
    out_ref[...] = e / jnp.sum(e, axis=-1, keepdims=True)


_sc_call = functools.partial(
    pl.kernel,
    mesh=plsc.VectorSubcoreMesh(core_axis_name="c", subcore_axis_name="s"),
    out_type=jax.ShapeDtypeStruct((R * 4 * L,), jnp.float32),
    scratch_types=[
        pltpu.VMEM((N,), jnp.float32),
        pltpu.VMEM((N,), jnp.float32),
        pltpu.VMEM((N,), jnp.float32),
        pltpu.VMEM((N,), jnp.float32),
        pltpu.VMEM((N,), jnp.float32),
        pltpu.VMEM((N,), jnp.float32),
        pltpu.VMEM((RPW * 4 * L,), jnp.float32),
        pltpu.SemaphoreType.DMA,
        pltpu.SemaphoreType.DMA,
    ],
)


@jax.jit
def _run(filtre, X, ids2, ids_sc, W_gcn, W_h, b_h, W_c, b_c):
    xw = pl.pallas_call(
        _xw_kernel,
        out_shape=jax.ShapeDtypeStruct((N, F), jnp.float32),
    )(X, W_gcn)
    xwt = xw.T

    partials = _sc_call(_sc_rows)(filtre, xwt)

    pooled_tc = pl.pallas_call(
        _body_kernel,
        grid=(M_BLOCKS,),
        in_specs=[
            pl.BlockSpec((N, F), lambda m: (0, 0)),
            pl.BlockSpec((BM, N), lambda m: (m, 0)),
            pl.BlockSpec((BM, 1), lambda m: (m, 0)),
        ],
        out_specs=pl.BlockSpec((F, G), lambda m: (0, 0)),
        out_shape=jax.ShapeDtypeStruct((F, G), jnp.float32),
        compiler_params=pltpu.CompilerParams(
            vmem_limit_bytes=64 * 1024 * 1024),
        scratch_shapes=[
            pltpu.VMEM((F, G), jnp.float32),
        ],
    )(xw, filtre, ids2)

    return pl.pallas_call(
        _merge_kernel,
        out_shape=jax.ShapeDtypeStruct((G, C), jnp.float32),
    )(pooled_tc, partials.reshape(R, 4 * L), ids_sc, W_h, b_h, W_c, b_c)


def kernel(filtre, X, node_indicator, W_gcn, W_h, b_h, W_c, b_c):
    ids2 = node_indicator.astype(jnp.int32).reshape(N, 1)
    return _run(filtre, X, ids2, ids2[S:], W_gcn, W_h,
                b_h.reshape(1, H), W_c, b_c.reshape(1, C))


# speedup vs baseline: 1.0750x; 1.0499x over previous
import functools

import jax
import jax.numpy as jnp
from jax import lax
from jax.experimental import pallas as pl
from jax.experimental.pallas import tpu as pltpu
from jax.experimental.pallas import tpu_sc as plsc

N = 10000
D = 128
F = 4
G = 64
H = 512
C = 10

NC, NS, L = 2, 16, 16
NW = NC * NS
R = 640
RPW = R // NW
S = N - R

BM = 672
M_BLOCKS = (S + BM - 1) // BM
UNROLL = 5
KCH5 = N // (L * UNROLL)


def _xwt_kernel(wg_ref, x_ref, xwt_ref):
    xwt_ref[...] = jax.lax.dot_general(
        wg_ref[...], x_ref[...], (((0,), (1,)), ((), ())),
        preferred_element_type=jnp.float32)


def _sc_rows(filt_hbm, xwt_hbm, out_hbm, bufa, bufb, xw0, xw1, xw2, xw3,
             outbuf, sema, semb):
    wid = lax.axis_index("s") * NC + lax.axis_index("c")
    base = S + wid * RPW
    pltpu.sync_copy(xwt_hbm.at[0], xw0)
    pltpu.sync_copy(xwt_hbm.at[1], xw1)
    pltpu.sync_copy(xwt_hbm.at[2], xw2)
    pltpu.sync_copy(xwt_hbm.at[3], xw3)

    def row_dot(buf, i):
        def chunk(ci, accs):
            a0, a1, a2, a3 = accs
            for u in range(UNROLL):
                off = ci * (L * UNROLL) + u * L
                rv = buf[pl.ds(off, L)]
                a0 = a0 + rv * xw0[pl.ds(off, L)]
                a1 = a1 + rv * xw1[pl.ds(off, L)]
                a2 = a2 + rv * xw2[pl.ds(off, L)]
                a3 = a3 + rv * xw3[pl.ds(off, L)]
            return (a0, a1, a2, a3)

        z = jnp.zeros((L,), jnp.float32)
        a0, a1, a2, a3 = lax.fori_loop(0, KCH5, chunk, (z, z, z, z))
        outbuf[pl.ds(i * 4 * L, L)] = a0
        outbuf[pl.ds(i * 4 * L + L, L)] = a1
        outbuf[pl.ds(i * 4 * L + 2 * L, L)] = a2
        outbuf[pl.ds(i * 4 * L + 3 * L, L)] = a3

    pltpu.async_copy(filt_hbm.at[base], bufa, sema)

    def pair(j, carry):
        r0 = base + 2 * j
        pltpu.async_copy(filt_hbm.at[r0 + 1], bufb, semb)
        pltpu.make_async_copy(filt_hbm.at[r0], bufa, sema).wait()
        row_dot(bufa, 2 * j)
        pltpu.async_copy(
            filt_hbm.at[jnp.minimum(r0 + 2, N - 1)], bufa, sema)
        pltpu.make_async_copy(filt_hbm.at[r0 + 1], bufb, semb).wait()
        row_dot(bufb, 2 * j + 1)
        return carry

    lax.fori_loop(0, RPW // 2, pair, 0)
    pltpu.make_async_copy(filt_hbm.at[0], bufa, sema).wait()
    pltpu.sync_copy(outbuf, out_hbm.at[pl.ds(wid * RPW * 4 * L, RPW * 4 * L)])


def _body_kernel(x_ref, wg_ref, filt_ref, ids_ref, pool_ref,
                 xw_ref, pooled_ref):
    m = pl.program_id(0)

    @pl.when(m == 0)
    def _init():
        xw_ref[...] = jnp.dot(x_ref[...], wg_ref[...],
                              preferred_element_type=jnp.float32)
        pooled_ref[...] = jnp.zeros_like(pooled_ref)

    h_blk = jnp.maximum(
        jnp.dot(filt_ref[...], xw_ref[...],
                preferred_element_type=jnp.float32), 0.0)
    gids = jax.lax.broadcasted_iota(jnp.int32, (BM, G), 1)
    rows = jax.lax.broadcasted_iota(jnp.int32, (BM, G), 0) + m * BM
    oh = (ids_ref[...] == gids) & (rows < N)
    cols = [jnp.max(jnp.where(oh, h_blk[:, f:f + 1], 0.0),
                    axis=0, keepdims=True) for f in range(F)]
    local = jnp.concatenate(cols, axis=0)
    pooled_ref[...] = jnp.maximum(pooled_ref[...], local)

    @pl.when(m == M_BLOCKS - 1)
    def _out():
        pool_ref[...] = pooled_ref[...]


def _merge_kernel(ptc_ref, part_ref, ids_ref, wh_ref, bh_ref,
                  wc_ref, bc_ref, out_ref):
    jrow = jax.lax.broadcasted_iota(jnp.int32, (4 * L, F), 0) // L
    jcol = jax.lax.broadcasted_iota(jnp.int32, (4 * L, F), 1)
    sel = jnp.where(jrow == jcol, 1.0, 0.0)
    h_sc = jnp.maximum(
        jnp.dot(part_ref[...], sel,
                preferred_element_type=jnp.float32), 0.0)
    gids = jax.lax.broadcasted_iota(jnp.int32, (R, G), 1)
    oh = ids_ref[...] == gids
    cols = [jnp.max(jnp.where(oh, h_sc[:, f:f + 1], 0.0),
                    axis=0, keepdims=True) for f in range(F)]
    local = jnp.concatenate(cols, axis=0)
    pooled_t = jnp.maximum(ptc_ref[...], local)
    z = jnp.maximum(
        jax.lax.dot_general(pooled_t, wh_ref[...],
                            (((0,), (0,)), ((), ())),
                            preferred_element_type=jnp.float32)
        + bh_ref[...], 0.0)
    logits = jnp.dot(z, wc_ref[...],
                     preferred_element_type=jnp.float32) + bc_ref[...]
    mx = jnp.max(logits, axis=-1, keepdims=True)
    e = jnp.exp(logits - mx)
    out_ref[...] = e / jnp.sum(e, axis=-1, keepdims=True)


_sc_call = functools.partial(
    pl.kernel,
    mesh=plsc.VectorSubcoreMesh(core_axis_name="c", subcore_axis_name="s"),
    out_type=jax.ShapeDtypeStruct((R * 4 * L,), jnp.float32),
    scratch_types=[
        pltpu.VMEM((N,), jnp.float32),
        pltpu.VMEM((N,), jnp.float32),
        pltpu.VMEM((N,), jnp.float32),
        pltpu.VMEM((N,), jnp.float32),
        pltpu.VMEM((N,), jnp.float32),
        pltpu.VMEM((N,), jnp.float32),
        pltpu.VMEM((RPW * 4 * L,), jnp.float32),
        pltpu.SemaphoreType.DMA,
        pltpu.SemaphoreType.DMA,
    ],
)


@jax.jit
def _run(filtre, X, node_indicator, W_gcn, W_h, b_h, W_c, b_c):
    ids2 = node_indicator.astype(jnp.int32).reshape(N, 1)
    ids_sc = jax.lax.slice(ids2, (S, 0), (N, 1))

    xwt = pl.pallas_call(
        _xwt_kernel,
        out_shape=jax.ShapeDtypeStruct((F, N), jnp.float32),
    )(W_gcn, X)

    partials = _sc_call(_sc_rows)(filtre, xwt)

    pooled_tc = pl.pallas_call(
        _body_kernel,
        grid=(M_BLOCKS,),
        in_specs=[
            pl.BlockSpec((N, D), lambda m: (0, 0)),
            pl.BlockSpec((D, F), lambda m: (0, 0)),
            pl.BlockSpec((BM, N), lambda m: (m, 0)),
            pl.BlockSpec((BM, 1), lambda m: (m, 0)),
        ],
        out_specs=pl.BlockSpec((F, G), lambda m: (0, 0)),
        out_shape=jax.ShapeDtypeStruct((F, G), jnp.float32),
        compiler_params=pltpu.CompilerParams(
            vmem_limit_bytes=64 * 1024 * 1024),
        scratch_shapes=[
            pltpu.VMEM((N, F), jnp.float32),
            pltpu.VMEM((F, G), jnp.float32),
        ],
    )(X, W_gcn, filtre, ids2)

    return pl.pallas_call(
        _merge_kernel,
        out_shape=jax.ShapeDtypeStruct((G, C), jnp.float32),
    )(pooled_tc, partials.reshape(R, 4 * L), ids_sc,
      W_h, b_h.reshape(1, H), W_c, b_c.reshape(1, C))


def kernel(filtre, X, node_indicator, W_gcn, W_h, b_h, W_c, b_c):
    return _run(filtre, X, node_indicator, W_gcn, W_h, b_h, W_c, b_c)


# speedup vs baseline: 1.2820x; 1.1926x over previous
import jax
import jax.numpy as jnp
from jax.experimental import pallas as pl
from jax.experimental.pallas import tpu as pltpu

N = 10000
D = 128
F = 4
G = 64
H = 512
C = 10

BM = 672
M_BLOCKS = (N + BM - 1) // BM


def _fused_kernel(x_ref, wg_ref, filt_ref, ids_ref, wh_ref, bh_ref,
                  wc_ref, bc_ref, out_ref, xw_ref, pooled_ref):
    m = pl.program_id(0)

    @pl.when(m == 0)
    def _init():
        xw_ref[...] = jnp.dot(x_ref[...], wg_ref[...],
                              preferred_element_type=jnp.float32)
        pooled_ref[...] = jnp.zeros_like(pooled_ref)

    h_blk = jnp.maximum(
        jnp.dot(filt_ref[...], xw_ref[...],
                preferred_element_type=jnp.float32), 0.0)

    gids = jax.lax.broadcasted_iota(jnp.int32, (BM, G), 1)
    rows = jax.lax.broadcasted_iota(jnp.int32, (BM, G), 0) + m * BM
    oh = (ids_ref[...] == gids) & (rows < N)
    cols = [jnp.max(jnp.where(oh, h_blk[:, f:f + 1], 0.0),
                    axis=0, keepdims=True) for f in range(F)]
    local = jnp.concatenate(cols, axis=0)
    pooled_ref[...] = jnp.maximum(pooled_ref[...], local)

    @pl.when(m == M_BLOCKS - 1)
    def _head():
        pooled_t = pooled_ref[...]
        z = jnp.maximum(
            jax.lax.dot_general(pooled_t, wh_ref[...],
                                (((0,), (0,)), ((), ())),
                                preferred_element_type=jnp.float32)
            + bh_ref[...], 0.0)
        logits = jnp.dot(z, wc_ref[...],
                         preferred_element_type=jnp.float32) + bc_ref[...]
        mx = jnp.max(logits, axis=-1, keepdims=True)
        e = jnp.exp(logits - mx)
        out_ref[...] = e / jnp.sum(e, axis=-1, keepdims=True)


@jax.jit
def _run(filtre, X, node_indicator, W_gcn, W_h, b_h, W_c, b_c):
    ids2 = node_indicator.astype(jnp.int32).reshape(N, 1)
    return pl.pallas_call(
        _fused_kernel,
        grid=(M_BLOCKS,),
        in_specs=[
            pl.BlockSpec((N, D), lambda m: (0, 0)),
            pl.BlockSpec((D, F), lambda m: (0, 0)),
            pl.BlockSpec((BM, N), lambda m: (m, 0)),
            pl.BlockSpec((BM, 1), lambda m: (m, 0)),
            pl.BlockSpec((F, H), lambda m: (0, 0)),
            pl.BlockSpec((1, H), lambda m: (0, 0)),
            pl.BlockSpec((H, C), lambda m: (0, 0)),
            pl.BlockSpec((1, C), lambda m: (0, 0)),
        ],
        out_specs=pl.BlockSpec((G, C), lambda m: (0, 0)),
        out_shape=jax.ShapeDtypeStruct((G, C), jnp.float32),
        compiler_params=pltpu.CompilerParams(
            vmem_limit_bytes=64 * 1024 * 1024),
        scratch_shapes=[
            pltpu.VMEM((N, F), jnp.float32),
            pltpu.VMEM((F, G), jnp.float32),
        ],
    )(X, W_gcn, filtre, ids2, W_h, b_h.reshape(1, H), W_c,
      b_c.reshape(1, C))


def kernel(filtre, X, node_indicator, W_gcn, W_h, b_h, W_c, b_c):
    return _run(filtre, X, node_indicator, W_gcn, W_h, b_h, W_c, b_c)
